# baseline (speedup 1.0000x reference)
import functools
import jax
import jax.numpy as jnp
from jax import lax
from jax.experimental import pallas as pl
from jax.experimental.pallas import tpu as pltpu
from jax.experimental.pallas import tpu_sc as plsc

P = 30000
L = 20
CIN = 14
C = 64
B = 2
NX = 512
NY = 512
NZ = 1
S = NZ * NY * NX
PX = 0.2
PY = 0.2
PZ = 8.0
XOFF = -51.1
YOFF = -51.1
ZOFF = -1.0
EPS = 1e-3

PPAD = 30720
TP = 640
TB = 1280
NLL = P * L


def _a_body(xt_ref, aux_ref, w_ref, m_ref, idx_ref, stats_ref):
    i = pl.program_id(0)
    xt = xt_ref[...]
    aux = aux_ref[...]
    npts = aux[:, 3:4]
    mask = aux[:, 5:6]
    centx = aux[:, 0:1] * PX + XOFF
    centy = aux[:, 1:2] * PY + YOFF
    centz = aux[:, 2:3] * PZ + ZOFF
    xyz = xt[:, :, 0:3]
    mean3 = jnp.sum(xyz, axis=0) / npts
    cent3 = jnp.concatenate([centx, centy, centz], axis=1)
    feat = jnp.concatenate(
        [xt[:, :, 0:4], xyz - mean3[None], xyz - cent3[None], xt], axis=2
    )
    feat = feat * mask[None, :, :]
    x2 = feat.reshape(L * TP, 24)
    h = jnp.dot(x2, w_ref[...], preferred_element_type=jnp.float32)
    h3 = h.reshape(L, TP, 128)
    m_ref[...] = jnp.max(h3, axis=0)
    ssum = jnp.sum(h, axis=0, keepdims=True)
    ssq = jnp.sum(h * h, axis=0, keepdims=True)
    st = jnp.concatenate([ssum, ssq, jnp.zeros((6, 128), jnp.float32)], axis=0)

    @pl.when(i == 0)
    def _():
        stats_ref[...] = jnp.zeros_like(stats_ref)

    stats_ref[...] += st

    ib = aux[:, 4:5].astype(jnp.int32)
    it = (
        aux[:, 2:3].astype(jnp.int32)
        + aux[:, 1:2].astype(jnp.int32) * NX
        + aux[:, 0:1].astype(jnp.int32)
    )
    valid = mask > 0.0
    i0 = jnp.where(valid, ib * S + it, B * S)
    i1 = jnp.where(valid, ib * (C * S) + it, B * C * S)
    colio = lax.broadcasted_iota(jnp.int32, (1, 128), 1)
    perch = jnp.where(colio < C, i1 + colio * S, B * C * S)
    idx_ref[...] = jnp.where(colio == C, i0, jnp.where(colio < C, perch, B * C * S))


def _b_body(m_ref, stats_ref, g_ref, bb_ref, wf_ref, f_ref, s_ref):
    st = stats_ref[...]
    mu = st[0:1, :] / NLL
    var = st[1:2, :] / NLL - mu * mu
    inv = lax.rsqrt(var + EPS) * g_ref[...]
    f = jnp.maximum((m_ref[...] - mu) * inv + bb_ref[...], 0.0)
    f_ref[...] = f
    s_ref[...] = jnp.dot(f, wf_ref[...], preferred_element_type=jnp.float32)


def _d_body(s1_ref, s2_ref, o_ref):
    i = pl.program_id(0)
    a1 = s1_ref[...]
    a2 = s2_ref[...]
    r = jnp.concatenate(
        [
            jnp.sum(a1, axis=0, keepdims=True),
            jnp.sum(a1 * a1, axis=0, keepdims=True),
            jnp.sum(a2, axis=0, keepdims=True),
            jnp.sum(a2 * a2, axis=0, keepdims=True),
            jnp.zeros((4, 1024), jnp.float32),
        ],
        axis=0,
    )

    @pl.when(i == 0)
    def _():
        o_ref[...] = jnp.zeros_like(o_ref)

    o_ref[...] += r


def _e_body(f_ref, s_ref, dst_ref, gb_ref, ct_ref):
    dst = dst_ref[...]
    n = float(B * S)
    mu1 = jnp.sum(dst[0:1, :]) / n
    var1 = jnp.sum(dst[1:2, :]) / n - mu1 * mu1
    mu2 = jnp.sum(dst[2:3, :]) / n
    var2 = jnp.sum(dst[3:4, :]) / n - mu2 * mu2
    gb = gb_ref[...]
    inv1 = lax.rsqrt(var1 + EPS) * gb[0, 0]
    inv2 = lax.rsqrt(var2 + EPS) * gb[0, 2]
    s = s_ref[...]
    w1 = (s[:, 0:1] - mu1) * inv1 + gb[0, 1]
    w2 = (s[:, 1:2] - mu2) * inv2 + gb[0, 3]
    a = jax.nn.sigmoid(w1 - w2)
    f = f_ref[...]
    comb = f[:, 0:C] * a + f[:, C : 2 * C] * (1.0 - a)
    rr = lax.broadcasted_iota(jnp.int32, (C, C), 0)
    cc = lax.broadcasted_iota(jnp.int32, (C, C), 1)
    ident = (rr == cc).astype(jnp.float32)
    ct_ref[...] = lax.dot_general(
        ident, comb, (((1,), (1,)), ((), ())),
        preferred_element_type=jnp.float32,
    )


NROW = PPAD // 16
RNG = (B * S) // 32
PIDSZ = B * S + 128
YSZ = B * C * S + 128
ZCH = 16384


def _w_body(idx_hbm, s1_hbm, s2_hbm, pid_hbm, sm1_hbm, sm2_hbm,
            idxv, valv, pidbuf, smbuf, sem):
    cc = lax.axis_index("c")
    ss = lax.axis_index("s")
    w = ss * 2 + cc
    base = w * RNG
    pltpu.sync_copy(idx_hbm, idxv)

    def fillp(i, _):
        pidbuf[pl.ds(i * 16, 16)] = jnp.full((16,), -1, jnp.int32)
        return 0

    lax.fori_loop(0, RNG // 16, fillp, 0)

    def scat_pid(i, _):
        v = idxv[pl.ds(i * 16, 16)]
        loc = v - base
        m = (loc >= 0) & (loc < RNG)
        lc = jnp.clip(loc, 0, RNG - 1)
        lanes = lax.iota(jnp.int32, 16)
        plsc.store_scatter(pidbuf, [lc], i * 16 + lanes, mask=m)
        return 0

    lax.fori_loop(0, NROW, scat_pid, 0)
    pltpu.sync_copy(pidbuf, pid_hbm.at[pl.ds(base, RNG)])

    for which in range(2):
        def fills(i, _):
            smbuf[pl.ds(i * 16, 16)] = jnp.zeros((16,), jnp.float32)
            return 0

        lax.fori_loop(0, RNG // 16, fills, 0)
        if which == 0:
            pltpu.sync_copy(s1_hbm, valv)
        else:
            pltpu.sync_copy(s2_hbm, valv)

        def scat_s(i, _):
            v = idxv[pl.ds(i * 16, 16)]
            loc = v - base
            m = (loc >= 0) & (loc < RNG)
            lc = jnp.clip(loc, 0, RNG - 1)
            plsc.store_scatter(smbuf, [lc], valv[pl.ds(i * 16, 16)], mask=m)
            return 0

        lax.fori_loop(0, NROW, scat_s, 0)
        if which == 0:
            pltpu.sync_copy(smbuf, sm1_hbm.at[pl.ds(base, RNG)])
        else:
            pltpu.sync_copy(smbuf, sm2_hbm.at[pl.ds(base, RNG)])


def _f_body(cell_hbm, idx_hbm, ct_hbm, pid_hbm, y_hbm,
            zbuf, cellv, valv, pidv, sem):
    cc = lax.axis_index("c")
    ss = lax.axis_index("s")
    w = ss * 2 + cc

    def fill(i, _):
        zbuf[pl.ds(i * 16, 16)] = jnp.zeros((16,), jnp.float32)
        return 0

    lax.fori_loop(0, ZCH // 16, fill, 0)

    def zero_chunk(k, _):
        b = k // (2 * (S // ZCH))
        r = k % (2 * (S // ZCH))
        ch = 2 * w + r // (S // ZCH)
        kk = r % (S // ZCH)
        off = (b * C + ch) * S + kk * ZCH
        pltpu.sync_copy(zbuf, y_hbm.at[pl.ds(off, ZCH)])
        return 0

    lax.fori_loop(0, 2 * 2 * (S // ZCH), zero_chunk, 0)

    pltpu.sync_copy(cell_hbm, cellv)
    pltpu.async_copy(pid_hbm.at[cellv], pidv, sem).wait()

    def winf(i, _):
        lanes = lax.iota(jnp.int32, 16)
        win = pidv[pl.ds(i * 16, 16)] == (i * 16 + lanes)
        pidv[pl.ds(i * 16, 16)] = jnp.where(win, 1, 0)
        return 0

    lax.fori_loop(0, NROW, winf, 0)

    for j in range(2):
        ch = 2 * w + j
        pltpu.sync_copy(idx_hbm.at[ch], cellv)

        def filt(i, _):
            v = cellv[pl.ds(i * 16, 16)]
            win = pidv[pl.ds(i * 16, 16)] > 0
            cellv[pl.ds(i * 16, 16)] = jnp.where(win, v, B * C * S)
            return 0

        lax.fori_loop(0, NROW, filt, 0)
        pltpu.sync_copy(ct_hbm.at[ch], valv)
        pltpu.async_copy(valv, y_hbm.at[cellv], sem).wait()


def kernel(pillars, W1, g1, b1, W2, g2, b2, wf1, gf1, bf1, wf2, gf2, bf2,
           coords_z, coords_y, coords_x, batch_idx, num_points):
    f32 = jnp.float32
    xt = jnp.transpose(pillars, (1, 0, 2))
    xt = jnp.pad(xt, ((0, 0), (0, PPAD - P), (0, 0)))
    aux = jnp.stack(
        [
            coords_x.astype(f32),
            coords_y.astype(f32),
            coords_z.astype(f32),
            num_points.astype(f32),
            batch_idx.astype(f32),
            jnp.ones((P,), f32),
        ],
        axis=1,
    )
    aux = jnp.concatenate([aux, jnp.zeros((P, 2), f32)], axis=1)
    aux = jnp.pad(aux, ((0, PPAD - P), (0, 0)))
    aux = aux.at[P:, 3].set(1.0)
    wboth = jnp.zeros((24, 128), f32)
    wboth = wboth.at[0:10, 0:C].set(W1)
    wboth = wboth.at[10:24, C : 2 * C].set(W2)

    grid_a = PPAD // TP
    m, idx, stats = pl.pallas_call(
        _a_body,
        grid=(grid_a,),
        in_specs=[
            pl.BlockSpec((L, TP, CIN), lambda i: (0, i, 0)),
            pl.BlockSpec((TP, 8), lambda i: (i, 0)),
            pl.BlockSpec((24, 128), lambda i: (0, 0)),
        ],
        out_specs=[
            pl.BlockSpec((TP, 128), lambda i: (i, 0)),
            pl.BlockSpec((TP, 128), lambda i: (i, 0)),
            pl.BlockSpec((8, 128), lambda i: (0, 0)),
        ],
        out_shape=[
            jax.ShapeDtypeStruct((PPAD, 128), f32),
            jax.ShapeDtypeStruct((PPAD, 128), jnp.int32),
            jax.ShapeDtypeStruct((8, 128), f32),
        ],
    )(xt, aux, wboth)

    gvec = jnp.concatenate([g1, g2]).reshape(1, 128)
    bvec = jnp.concatenate([b1, b2]).reshape(1, 128)
    wfb = jnp.zeros((128, 2), f32)
    wfb = wfb.at[0:C, 0].set(wf1)
    wfb = wfb.at[C : 2 * C, 1].set(wf2)

    grid_b = PPAD // TB
    f, s12 = pl.pallas_call(
        _b_body,
        grid=(grid_b,),
        in_specs=[
            pl.BlockSpec((TB, 128), lambda i: (i, 0)),
            pl.BlockSpec((8, 128), lambda i: (0, 0)),
            pl.BlockSpec((1, 128), lambda i: (0, 0)),
            pl.BlockSpec((1, 128), lambda i: (0, 0)),
            pl.BlockSpec((128, 2), lambda i: (0, 0)),
        ],
        out_specs=[
            pl.BlockSpec((TB, 128), lambda i: (i, 0)),
            pl.BlockSpec((TB, 2), lambda i: (i, 0)),
        ],
        out_shape=[
            jax.ShapeDtypeStruct((PPAD, 128), f32),
            jax.ShapeDtypeStruct((PPAD, 2), f32),
        ],
    )(m, stats, gvec, bvec, wfb)

    mesh = plsc.VectorSubcoreMesh(core_axis_name="c", subcore_axis_name="s")
    e = idx[:, C].reshape(NROW, 16)
    li = jnp.arange(16)
    later = li[None, :, None] < li[None, None, :]
    dup = jnp.any((e[:, :, None] == e[:, None, :]) & later, axis=2)
    idx_sm = jnp.where(dup.reshape(PPAD), B * S, idx[:, C])
    s1v = s12[:, 0]
    s2v = s12[:, 1]
    pid, smap1, smap2 = pl.kernel(
        _w_body,
        out_type=[
            jax.ShapeDtypeStruct((PIDSZ,), jnp.int32),
            jax.ShapeDtypeStruct((B * S,), f32),
            jax.ShapeDtypeStruct((B * S,), f32),
        ],
        mesh=mesh,
        compiler_params=pltpu.CompilerParams(needs_layout_passes=False),
        scratch_types=[
            pltpu.VMEM((PPAD,), jnp.int32),
            pltpu.VMEM((PPAD,), f32),
            pltpu.VMEM((RNG,), jnp.int32),
            pltpu.VMEM((RNG,), f32),
            pltpu.SemaphoreType.DMA,
        ],
    )(idx_sm, s1v, s2v)

    sm1 = smap1.reshape(B * S // 1024, 1024)
    sm2 = smap2.reshape(B * S // 1024, 1024)
    TD = 64
    grid_d = (B * S // 1024) // TD
    dstats = pl.pallas_call(
        _d_body,
        grid=(grid_d,),
        in_specs=[
            pl.BlockSpec((TD, 1024), lambda i: (i, 0)),
            pl.BlockSpec((TD, 1024), lambda i: (i, 0)),
        ],
        out_specs=pl.BlockSpec((8, 1024), lambda i: (0, 0)),
        out_shape=jax.ShapeDtypeStruct((8, 1024), f32),
    )(sm1, sm2)

    gbv = jnp.stack([gf1[0], bf1[0], gf2[0], bf2[0]]).reshape(1, 4)
    gbv = jnp.concatenate([gbv, jnp.zeros((1, 124), f32)], axis=1)
    combt = pl.pallas_call(
        _e_body,
        grid=(grid_b,),
        in_specs=[
            pl.BlockSpec((TB, 128), lambda i: (i, 0)),
            pl.BlockSpec((TB, 2), lambda i: (i, 0)),
            pl.BlockSpec((8, 1024), lambda i: (0, 0)),
            pl.BlockSpec((1, 128), lambda i: (0, 0)),
        ],
        out_specs=pl.BlockSpec((C, TB), lambda i: (0, i)),
        out_shape=jax.ShapeDtypeStruct((C, PPAD), f32),
    )(f, s12, dstats, gbv)

    idxt = jnp.transpose(idx[:, :C])
    y1 = pl.kernel(
        _f_body,
        out_type=jax.ShapeDtypeStruct((YSZ,), f32),
        mesh=mesh,
        compiler_params=pltpu.CompilerParams(needs_layout_passes=False),
        scratch_types=[
            pltpu.VMEM((ZCH,), f32),
            pltpu.VMEM((PPAD,), jnp.int32),
            pltpu.VMEM((PPAD,), f32),
            pltpu.VMEM((PPAD,), jnp.int32),
            pltpu.SemaphoreType.DMA,
        ],
    )(idx_sm, idxt, combt, pid)
    return y1[: B * C * S].reshape(B, C * NZ, NY, NX)


# speedup vs baseline: 12.9905x; 12.9905x over previous
import functools
import jax
import jax.numpy as jnp
from jax import lax
from jax.experimental import pallas as pl
from jax.experimental.pallas import tpu as pltpu
from jax.experimental.pallas import tpu_sc as plsc

P = 30000
L = 20
CIN = 14
C = 64
B = 2
NX = 512
NY = 512
NZ = 1
S = NZ * NY * NX
PX = 0.2
PY = 0.2
PZ = 8.0
XOFF = -51.1
YOFF = -51.1
ZOFF = -1.0
EPS = 1e-3

PPAD = 30720
TP = 640
TB = 1280
NLL = P * L


def _a_body(xt_ref, aux_ref, w_ref, m_ref, idx_ref, stats_ref):
    i = pl.program_id(0)
    xt = xt_ref[...]
    aux = aux_ref[...]
    npts = aux[:, 3:4]
    mask = aux[:, 5:6]
    centx = aux[:, 0:1] * PX + XOFF
    centy = aux[:, 1:2] * PY + YOFF
    centz = aux[:, 2:3] * PZ + ZOFF
    xyz = xt[:, :, 0:3]
    mean3 = jnp.sum(xyz, axis=0) / npts
    cent3 = jnp.concatenate([centx, centy, centz], axis=1)
    feat = jnp.concatenate(
        [xt[:, :, 0:4], xyz - mean3[None], xyz - cent3[None], xt], axis=2
    )
    feat = feat * mask[None, :, :]
    x2 = feat.reshape(L * TP, 24)
    h = jnp.dot(x2, w_ref[...], preferred_element_type=jnp.float32)
    h3 = h.reshape(L, TP, 128)
    m_ref[...] = jnp.max(h3, axis=0)
    ssum = jnp.sum(h, axis=0, keepdims=True)
    ssq = jnp.sum(h * h, axis=0, keepdims=True)
    st = jnp.concatenate([ssum, ssq, jnp.zeros((6, 128), jnp.float32)], axis=0)

    @pl.when(i == 0)
    def _():
        stats_ref[...] = jnp.zeros_like(stats_ref)

    stats_ref[...] += st

    ib = aux[:, 4:5].astype(jnp.int32)
    it = (
        aux[:, 2:3].astype(jnp.int32)
        + aux[:, 1:2].astype(jnp.int32) * NX
        + aux[:, 0:1].astype(jnp.int32)
    )
    valid = mask > 0.0
    i0 = jnp.where(valid, ib * S + it, B * S)
    i1 = jnp.where(valid, ib * (C * S) + it, B * C * S)
    colio = lax.broadcasted_iota(jnp.int32, (1, 128), 1)
    perch = jnp.where(colio < C, i1 + colio * S, B * C * S)
    idx_ref[...] = jnp.where(colio == C, i0, jnp.where(colio < C, perch, B * C * S))


def _b_body(m_ref, stats_ref, g_ref, bb_ref, wf_ref, f_ref, s_ref):
    st = stats_ref[...]
    mu = st[0:1, :] / NLL
    var = st[1:2, :] / NLL - mu * mu
    inv = lax.rsqrt(var + EPS) * g_ref[...]
    f = jnp.maximum((m_ref[...] - mu) * inv + bb_ref[...], 0.0)
    f_ref[...] = f
    s_ref[...] = jnp.dot(f, wf_ref[...], preferred_element_type=jnp.float32)


def _d_body(s1_ref, s2_ref, o_ref):
    i = pl.program_id(0)
    a1 = s1_ref[...]
    a2 = s2_ref[...]
    r = jnp.concatenate(
        [
            jnp.sum(a1, axis=0, keepdims=True),
            jnp.sum(a1 * a1, axis=0, keepdims=True),
            jnp.sum(a2, axis=0, keepdims=True),
            jnp.sum(a2 * a2, axis=0, keepdims=True),
            jnp.zeros((4, 1024), jnp.float32),
        ],
        axis=0,
    )

    @pl.when(i == 0)
    def _():
        o_ref[...] = jnp.zeros_like(o_ref)

    o_ref[...] += r


def _e_body(f_ref, s_ref, dst_ref, gb_ref, ct_ref):
    dst = dst_ref[...]
    n = float(B * S)
    mu1 = jnp.sum(dst[0:1, :]) / n
    var1 = jnp.sum(dst[1:2, :]) / n - mu1 * mu1
    mu2 = jnp.sum(dst[2:3, :]) / n
    var2 = jnp.sum(dst[3:4, :]) / n - mu2 * mu2
    gb = gb_ref[...]
    inv1 = lax.rsqrt(var1 + EPS) * gb[0, 0]
    inv2 = lax.rsqrt(var2 + EPS) * gb[0, 2]
    s = s_ref[...]
    w1 = (s[:, 0:1] - mu1) * inv1 + gb[0, 1]
    w2 = (s[:, 1:2] - mu2) * inv2 + gb[0, 3]
    a = jax.nn.sigmoid(w1 - w2)
    f = f_ref[...]
    comb = f[:, 0:C] * a + f[:, C : 2 * C] * (1.0 - a)
    ct_ref[...] = jnp.concatenate(
        [comb, jnp.zeros((comb.shape[0], 128 - C), jnp.float32)], axis=1
    )


NROW = PPAD // 16
RNG = (B * S) // 32
PIDSZ = B * S + 128
YSZ = B * C * S + 128
ZCH = 16384


def _w_body(idx_hbm, s1_hbm, s2_hbm, pid_hbm, sm1_hbm, sm2_hbm,
            idxv, valv, pidbuf, smbuf, sem):
    cc = lax.axis_index("c")
    ss = lax.axis_index("s")
    w = ss * 2 + cc
    base = w * RNG
    pltpu.sync_copy(idx_hbm, idxv)

    def fillp(i, _):
        pidbuf[pl.ds(i * 16, 16)] = jnp.full((16,), -1, jnp.int32)
        return 0

    lax.fori_loop(0, RNG // 16, fillp, 0)

    def scat_pid(i, _):
        v = idxv[pl.ds(i * 16, 16)]
        loc = v - base
        m = (loc >= 0) & (loc < RNG)
        lc = jnp.clip(loc, 0, RNG - 1)
        lanes = lax.iota(jnp.int32, 16)
        plsc.store_scatter(pidbuf, [lc], i * 16 + lanes, mask=m)
        return 0

    lax.fori_loop(0, NROW, scat_pid, 0)
    pltpu.sync_copy(pidbuf, pid_hbm.at[pl.ds(base, RNG)])

    for which in range(2):
        def fills(i, _):
            smbuf[pl.ds(i * 16, 16)] = jnp.zeros((16,), jnp.float32)
            return 0

        lax.fori_loop(0, RNG // 16, fills, 0)
        if which == 0:
            pltpu.sync_copy(s1_hbm, valv)
        else:
            pltpu.sync_copy(s2_hbm, valv)

        def scat_s(i, _):
            v = idxv[pl.ds(i * 16, 16)]
            loc = v - base
            m = (loc >= 0) & (loc < RNG)
            lc = jnp.clip(loc, 0, RNG - 1)
            plsc.store_scatter(smbuf, [lc], valv[pl.ds(i * 16, 16)], mask=m)
            return 0

        lax.fori_loop(0, NROW, scat_s, 0)
        if which == 0:
            pltpu.sync_copy(smbuf, sm1_hbm.at[pl.ds(base, RNG)])
        else:
            pltpu.sync_copy(smbuf, sm2_hbm.at[pl.ds(base, RNG)])


PSUB = PPAD // 32
TS = 16384


def _fp_body(cell_hbm, comb_hbm, pid_hbm, z_hbm, cellv, pidv, rowsv, sem):
    cc = lax.axis_index("c")
    ss = lax.axis_index("s")
    w = ss * 2 + cc
    base = w * PSUB
    pltpu.sync_copy(cell_hbm.at[pl.ds(base, PSUB)], cellv)
    pltpu.async_copy(pid_hbm.at[cellv], pidv, sem).wait()

    def winf(i, _):
        lanes = lax.iota(jnp.int32, 16)
        v = cellv[pl.ds(i * 16, 16)]
        win = pidv[pl.ds(i * 16, 16)] == (base + i * 16 + lanes)
        cellv[pl.ds(i * 16, 16)] = jnp.where(win, v, B * S)
        return 0

    lax.fori_loop(0, PSUB // 16, winf, 0)
    pltpu.sync_copy(comb_hbm.at[pl.ds(base, PSUB)], rowsv)
    pltpu.async_copy(rowsv, z_hbm.at[cellv], sem).wait()


def _g_body(z_ref, pid_ref, y_ref):
    pm = pid_ref[...] >= 0
    zm = jnp.where(pm, z_ref[:, 0:C], 0.0)
    rr = lax.broadcasted_iota(jnp.int32, (C, C), 0)
    ncc = lax.broadcasted_iota(jnp.int32, (C, C), 1)
    ident = (rr == ncc).astype(jnp.float32)
    yt = lax.dot_general(
        ident, zm, (((1,), (1,)), ((), ())),
        preferred_element_type=jnp.float32,
    )
    y_ref[...] = yt[None]


def kernel(pillars, W1, g1, b1, W2, g2, b2, wf1, gf1, bf1, wf2, gf2, bf2,
           coords_z, coords_y, coords_x, batch_idx, num_points):
    f32 = jnp.float32
    xt = jnp.transpose(pillars, (1, 0, 2))
    xt = jnp.pad(xt, ((0, 0), (0, PPAD - P), (0, 0)))
    aux = jnp.stack(
        [
            coords_x.astype(f32),
            coords_y.astype(f32),
            coords_z.astype(f32),
            num_points.astype(f32),
            batch_idx.astype(f32),
            jnp.ones((P,), f32),
        ],
        axis=1,
    )
    aux = jnp.concatenate([aux, jnp.zeros((P, 2), f32)], axis=1)
    aux = jnp.pad(aux, ((0, PPAD - P), (0, 0)))
    aux = aux.at[P:, 3].set(1.0)
    wboth = jnp.zeros((24, 128), f32)
    wboth = wboth.at[0:10, 0:C].set(W1)
    wboth = wboth.at[10:24, C : 2 * C].set(W2)

    grid_a = PPAD // TP
    m, idx, stats = pl.pallas_call(
        _a_body,
        grid=(grid_a,),
        in_specs=[
            pl.BlockSpec((L, TP, CIN), lambda i: (0, i, 0)),
            pl.BlockSpec((TP, 8), lambda i: (i, 0)),
            pl.BlockSpec((24, 128), lambda i: (0, 0)),
        ],
        out_specs=[
            pl.BlockSpec((TP, 128), lambda i: (i, 0)),
            pl.BlockSpec((TP, 128), lambda i: (i, 0)),
            pl.BlockSpec((8, 128), lambda i: (0, 0)),
        ],
        out_shape=[
            jax.ShapeDtypeStruct((PPAD, 128), f32),
            jax.ShapeDtypeStruct((PPAD, 128), jnp.int32),
            jax.ShapeDtypeStruct((8, 128), f32),
        ],
    )(xt, aux, wboth)

    gvec = jnp.concatenate([g1, g2]).reshape(1, 128)
    bvec = jnp.concatenate([b1, b2]).reshape(1, 128)
    wfb = jnp.zeros((128, 2), f32)
    wfb = wfb.at[0:C, 0].set(wf1)
    wfb = wfb.at[C : 2 * C, 1].set(wf2)

    grid_b = PPAD // TB
    f, s12 = pl.pallas_call(
        _b_body,
        grid=(grid_b,),
        in_specs=[
            pl.BlockSpec((TB, 128), lambda i: (i, 0)),
            pl.BlockSpec((8, 128), lambda i: (0, 0)),
            pl.BlockSpec((1, 128), lambda i: (0, 0)),
            pl.BlockSpec((1, 128), lambda i: (0, 0)),
            pl.BlockSpec((128, 2), lambda i: (0, 0)),
        ],
        out_specs=[
            pl.BlockSpec((TB, 128), lambda i: (i, 0)),
            pl.BlockSpec((TB, 2), lambda i: (i, 0)),
        ],
        out_shape=[
            jax.ShapeDtypeStruct((PPAD, 128), f32),
            jax.ShapeDtypeStruct((PPAD, 2), f32),
        ],
    )(m, stats, gvec, bvec, wfb)

    mesh = plsc.VectorSubcoreMesh(core_axis_name="c", subcore_axis_name="s")
    e = idx[:, C].reshape(NROW, 16)
    li = jnp.arange(16)
    later = li[None, :, None] < li[None, None, :]
    dup = jnp.any((e[:, :, None] == e[:, None, :]) & later, axis=2)
    idx_sm = jnp.where(dup.reshape(PPAD), B * S, idx[:, C])
    s1v = s12[:, 0]
    s2v = s12[:, 1]
    pid, smap1, smap2 = pl.kernel(
        _w_body,
        out_type=[
            jax.ShapeDtypeStruct((PIDSZ,), jnp.int32),
            jax.ShapeDtypeStruct((B * S,), f32),
            jax.ShapeDtypeStruct((B * S,), f32),
        ],
        mesh=mesh,
        compiler_params=pltpu.CompilerParams(needs_layout_passes=False),
        scratch_types=[
            pltpu.VMEM((PPAD,), jnp.int32),
            pltpu.VMEM((PPAD,), f32),
            pltpu.VMEM((RNG,), jnp.int32),
            pltpu.VMEM((RNG,), f32),
            pltpu.SemaphoreType.DMA,
        ],
    )(idx_sm, s1v, s2v)

    sm1 = smap1.reshape(B * S // 1024, 1024)
    sm2 = smap2.reshape(B * S // 1024, 1024)
    TD = 64
    grid_d = (B * S // 1024) // TD
    dstats = pl.pallas_call(
        _d_body,
        grid=(grid_d,),
        in_specs=[
            pl.BlockSpec((TD, 1024), lambda i: (i, 0)),
            pl.BlockSpec((TD, 1024), lambda i: (i, 0)),
        ],
        out_specs=pl.BlockSpec((8, 1024), lambda i: (0, 0)),
        out_shape=jax.ShapeDtypeStruct((8, 1024), f32),
    )(sm1, sm2)

    gbv = jnp.stack([gf1[0], bf1[0], gf2[0], bf2[0]]).reshape(1, 4)
    gbv = jnp.concatenate([gbv, jnp.zeros((1, 124), f32)], axis=1)
    combt = pl.pallas_call(
        _e_body,
        grid=(grid_b,),
        in_specs=[
            pl.BlockSpec((TB, 128), lambda i: (i, 0)),
            pl.BlockSpec((TB, 2), lambda i: (i, 0)),
            pl.BlockSpec((8, 1024), lambda i: (0, 0)),
            pl.BlockSpec((1, 128), lambda i: (0, 0)),
        ],
        out_specs=pl.BlockSpec((TB, 128), lambda i: (i, 0)),
        out_shape=jax.ShapeDtypeStruct((PPAD, 128), f32),
    )(f, s12, dstats, gbv)

    z = pl.kernel(
        _fp_body,
        out_type=jax.ShapeDtypeStruct((B * S + 8, 128), f32),
        mesh=mesh,
        compiler_params=pltpu.CompilerParams(needs_layout_passes=False),
        scratch_types=[
            pltpu.VMEM((PSUB,), jnp.int32),
            pltpu.VMEM((PSUB,), jnp.int32),
            pltpu.VMEM((PSUB, 128), f32),
            pltpu.SemaphoreType.DMA,
        ],
    )(idx_sm, combt, pid)

    pidcol = pid[: B * S].reshape(B * S, 1)
    y = pl.pallas_call(
        _g_body,
        grid=(B, S // TS),
        in_specs=[
            pl.BlockSpec((TS, 128), lambda b, i: (b * (S // TS) + i, 0)),
            pl.BlockSpec((TS, 1), lambda b, i: (b * (S // TS) + i, 0)),
        ],
        out_specs=pl.BlockSpec((1, C, TS), lambda b, i: (b, 0, i)),
        out_shape=jax.ShapeDtypeStruct((B, C, S), f32),
    )(z, pidcol)
    return y.reshape(B, C * NZ, NY, NX)
